# baseline (speedup 1.0000x reference)
import functools

import jax
import jax.numpy as jnp
from jax import lax
from jax.experimental import pallas as pl
from jax.experimental.pallas import tpu as pltpu
from jax.experimental.pallas import tpu_sc as plsc

_N = 10000
_E = 320000
_D = 128
_HW = _D // 2
_NC = 2
_NS = 16
_RB = 80
_SEG = 5
_SB = 50
_RING = 5
_CH = 640
_CHL = _N - (_NS - 1) * _CH
_EB = 3200

_TC_PARAMS = pltpu.CompilerParams(vmem_limit_bytes=100 * 2**20)
_SC_PARAMS = pltpu.CompilerParams(needs_layout_passes=False,
                                  use_tc_tiling_on_sc=False)


def _row_dot(v_ref, m):
    return lax.dot_general(v_ref[...], m,
                           dimension_numbers=(((0,), (1,)), ((), ())),
                           preferred_element_type=jnp.float32)


def _node_prologue(x, W_in, b_in, W_res, b_res, W1, a1s, a1d):
    def body(x_ref, wi_ref, bi_ref, wr_ref, br_ref, w1_ref, as_ref, ad_ref,
             ht2_ref, ss_ref, sd_ref):
        h = jnp.dot(x_ref[...], wi_ref[...],
                    preferred_element_type=jnp.float32) + bi_ref[...]
        h = h + jnp.maximum(
            jnp.dot(h, wr_ref[...], preferred_element_type=jnp.float32)
            + br_ref[...], 0.0)
        ht = jnp.dot(h, w1_ref[...], preferred_element_type=jnp.float32)
        ht2_ref[pl.ds(0, _N), :] = ht[:, :_HW]
        ht2_ref[pl.ds(_N, _N), :] = ht[:, _HW:]
        ss_ref[...] = _row_dot(as_ref, ht)
        sd_ref[...] = _row_dot(ad_ref, ht)

    return pl.pallas_call(
        body,
        out_shape=[jax.ShapeDtypeStruct((2 * _N, _HW), jnp.float32),
                   jax.ShapeDtypeStruct((1, _N), jnp.float32),
                   jax.ShapeDtypeStruct((1, _N), jnp.float32)],
        compiler_params=_TC_PARAMS,
    )(x, W_in, b_in.reshape(1, _D), W_res, b_res.reshape(1, _D), W1, a1s, a1d)


def _edge_dense1(edge_attr, W_edge, We1, v1):
    def body(ea_ref, we_ref, w1_ref, v1_ref, e1_ref, s1_ref):
        ea0 = jnp.maximum(
            jnp.dot(ea_ref[...], we_ref[...], preferred_element_type=jnp.float32),
            0.0)
        e1 = jnp.dot(ea0, w1_ref[...], preferred_element_type=jnp.float32)
        e1_ref[...] = e1
        s1_ref[...] = _row_dot(v1_ref, e1).reshape(1, 1, _EB)

    full = lambda i: (0, 0)
    blk = lambda i: (i, 0)
    return pl.pallas_call(
        body,
        grid=(_E // _EB,),
        in_specs=[pl.BlockSpec((_EB, 16), blk),
                  pl.BlockSpec((16, _D), full),
                  pl.BlockSpec((_D, _D), full),
                  pl.BlockSpec((_D, 1), full)],
        out_specs=[pl.BlockSpec((_EB, _D), blk),
                   pl.BlockSpec((1, 1, _EB), lambda i: (i, 0, 0))],
        out_shape=[jax.ShapeDtypeStruct((_E, _D), jnp.float32),
                   jax.ShapeDtypeStruct((_E // _EB, 1, _EB), jnp.float32)],
        compiler_params=_TC_PARAMS,
    )(edge_attr, W_edge, We1, v1)


def _edge_dense2(e1, We2, v2):
    def body(e1_ref, w2_ref, v2_ref, e2_ref, s2_ref):
        e2 = jnp.dot(e1_ref[...], w2_ref[...], preferred_element_type=jnp.float32)
        e2_ref[...] = e2
        s2_ref[...] = _row_dot(v2_ref, e2).reshape(1, 1, _EB)

    full = lambda i: (0, 0)
    blk = lambda i: (i, 0)
    return pl.pallas_call(
        body,
        grid=(_E // _EB,),
        in_specs=[pl.BlockSpec((_EB, _D), blk),
                  pl.BlockSpec((_D, _D), full),
                  pl.BlockSpec((_D, 1), full)],
        out_specs=[pl.BlockSpec((_EB, _D), blk),
                   pl.BlockSpec((1, 1, _EB), lambda i: (i, 0, 0))],
        out_shape=[jax.ShapeDtypeStruct((_E, _D), jnp.float32),
                   jax.ShapeDtypeStruct((_E // _EB, 1, _EB), jnp.float32)],
        compiler_params=_TC_PARAMS,
    )(e1, We2, v2)


def _elu(x):
    return jnp.where(x > 0, x, jnp.exp(jnp.minimum(x, 0.0)) - 1.0)


def _combine_mid(wp, dp, W2, a2s, a2d):
    def body(wp_ref, dp_ref, w2_ref, as_ref, ad_ref, ht2_ref, ss_ref, sd_ref):
        w = jnp.concatenate([wp_ref[0], wp_ref[1]], axis=1)
        h1 = _elu(w / (dp_ref[...] + 1e-16))
        ht = jnp.dot(h1, w2_ref[...], preferred_element_type=jnp.float32)
        ht2_ref[pl.ds(0, _N), :] = ht[:, :_HW]
        ht2_ref[pl.ds(_N, _N), :] = ht[:, _HW:]
        ss_ref[...] = _row_dot(as_ref, ht)
        sd_ref[...] = _row_dot(ad_ref, ht)

    return pl.pallas_call(
        body,
        out_shape=[jax.ShapeDtypeStruct((2 * _N, _HW), jnp.float32),
                   jax.ShapeDtypeStruct((1, _N), jnp.float32),
                   jax.ShapeDtypeStruct((1, _N), jnp.float32)],
        compiler_params=_TC_PARAMS,
    )(wp, dp, W2, a2s, a2d)


def _combine_final(wp, dp):
    def body(wp_ref, dp_ref, h_ref):
        w = jnp.concatenate([wp_ref[0], wp_ref[1]], axis=1)
        h_ref[...] = _elu(w / (dp_ref[...] + 1e-16))

    return pl.pallas_call(
        body,
        out_shape=jax.ShapeDtypeStruct((_N, _D), jnp.float32),
        compiler_params=_TC_PARAMS,
    )(wp, dp)


def _sc_gat(ht2, ss, sd, se4, src4, dst4):
    mesh = plsc.VectorSubcoreMesh(core_axis_name="c", subcore_axis_name="s")

    @functools.partial(
        pl.kernel,
        out_type=[jax.ShapeDtypeStruct((_NC, _N, _HW), jnp.float32),
                  jax.ShapeDtypeStruct((_N,), jnp.float32)],
        mesh=mesh,
        compiler_params=_SC_PARAMS,
        scratch_types=(
            [pltpu.VMEM((_N,), jnp.float32),
             pltpu.VMEM((_N,), jnp.float32),
             pltpu.VMEM((_SB, _RB), jnp.int32),
             pltpu.VMEM((_SB, _RB), jnp.int32),
             pltpu.VMEM((_SB, _RB), jnp.float32),
             pltpu.VMEM((_SB, _RB), jnp.float32)]
            + [pltpu.VMEM((_RB, _HW), jnp.float32)] * _RING
            + [pltpu.VMEM_SHARED((_N, _HW), jnp.float32),
               pltpu.VMEM_SHARED((_N,), jnp.float32)]
            + [pltpu.SemaphoreType.DMA] * (2 * _RING)),
    )
    def k(ht_hbm, ss_hbm, sd_hbm, se_hbm, src_hbm, dst_hbm,
          wsum_hbm, den_hbm,
          ss_tab, sd_tab, src_seg, dst_seg, se_seg, w_seg,
          g0, g1, g2, g3, g4, wsum_sh, den_sh,
          gs0, gs1, gs2, gs3, gs4, ts0, ts1, ts2, ts3, ts4):
        c = lax.axis_index("c")
        s = lax.axis_index("s")
        gbufs = (g0, g1, g2, g3, g4)
        gsems = (gs0, gs1, gs2, gs3, gs4)
        ssems = (ts0, ts1, ts2, ts3, ts4)

        row0 = pl.multiple_of(s * _CH, 8)

        def _per_range(fn):
            @pl.when(s < _NS - 1)
            def _():
                fn(_CH)

            @pl.when(s == _NS - 1)
            def _():
                fn(_CHL)

        z16 = jnp.zeros((16,), jnp.float32)

        @pl.loop(0, _RB)
        def _(j):
            for q in range(_HW // 16):
                g0[j, pl.ds(q * 16, 16)] = z16

        def _zero(n):
            for t in range(n // _RB):
                pltpu.sync_copy(g0, wsum_sh.at[pl.ds(row0 + t * _RB, _RB)])

                @pl.when(c == 0)
                def _():
                    pltpu.sync_copy(g0.at[0, pl.ds(0, _RB)],
                                    den_sh.at[pl.ds(row0 + t * _RB, _RB)])

        _per_range(_zero)
        pltpu.sync_copy(ss_hbm, ss_tab)
        pltpu.sync_copy(sd_hbm, sd_tab)
        plsc.subcore_barrier()

        off = c * _N

        @pl.loop(0, _SEG)
        def _(seg):
            pltpu.sync_copy(src_hbm.at[s, seg], src_seg)
            pltpu.sync_copy(dst_hbm.at[s, seg], dst_seg)
            pltpu.sync_copy(se_hbm.at[s, seg], se_seg)

            @pl.loop(0, _SB)
            def _(b):
                for g in range(_RB // 16):
                    sl = pl.ds(g * 16, 16)
                    si = src_seg[b, sl]
                    di = dst_seg[b, sl]
                    vs = plsc.load_gather(ss_tab, [si])
                    vd = plsc.load_gather(sd_tab, [di])
                    l = vs + vd + se_seg[b, sl]
                    l = jnp.where(l >= 0.0, l, 0.2 * l)
                    w_seg[b, sl] = jnp.exp(l)
                    src_seg[b, sl] = si + off

                @pl.when(c == 0)
                def _():
                    pltpu.sync_copy(w_seg.at[b], den_sh.at[dst_seg.at[b]],
                                    add=True)

            for p in range(_RING - 1):
                pltpu.async_copy(ht_hbm.at[src_seg.at[p]], gbufs[p], gsems[p])

            @pl.loop(0, _SB, step=_RING)
            def _(r):
                for p in range(_RING):
                    rr = r + p
                    gbuf, gsem, ssem = gbufs[p], gsems[p], ssems[p]
                    pm = (p + _RING - 1) % _RING
                    pltpu.make_async_copy(ht_hbm.at[src_seg.at[rr]],
                                          gbuf, gsem).wait()

                    for j16 in range(_RB // 16):
                        w16 = w_seg[rr, pl.ds(j16 * 16, 16)]
                        for jj in range(16):
                            wj = w16[jj]
                            row = j16 * 16 + jj
                            for q in range(_HW // 16):
                                sl2 = pl.ds(q * 16, 16)
                                gbuf[row, sl2] = gbuf[row, sl2] * wj

                    pltpu.async_copy(gbuf, wsum_sh.at[dst_seg.at[rr]],
                                     ssem, add=True)

                    @pl.when(rr >= 1)
                    def _():
                        pltpu.make_async_copy(
                            gbufs[pm], wsum_sh.at[dst_seg.at[rr - 1]],
                            ssems[pm]).wait()

                    @pl.when(rr + _RING - 1 < _SB)
                    def _():
                        pltpu.async_copy(ht_hbm.at[src_seg.at[rr + _RING - 1]],
                                         gbufs[pm], gsems[pm])

            pltpu.make_async_copy(gbufs[(_SB - 1) % _RING],
                                  wsum_sh.at[dst_seg.at[_SB - 1]],
                                  ssems[(_SB - 1) % _RING]).wait()

        plsc.subcore_barrier()

        def _publish(n):
            for t in range(n // _RB):
                gb = gbufs[t % 2]
                r0 = row0 + t * _RB
                pltpu.sync_copy(wsum_sh.at[pl.ds(r0, _RB)], gb)
                pltpu.sync_copy(gb, wsum_hbm.at[c, pl.ds(r0, _RB)])

                @pl.when(c == 0)
                def _():
                    pltpu.sync_copy(den_sh.at[pl.ds(r0, _RB)],
                                    gb.at[0, pl.ds(0, _RB)])
                    pltpu.sync_copy(gb.at[0, pl.ds(0, _RB)],
                                    den_hbm.at[pl.ds(r0, _RB)])

        _per_range(_publish)

    return k(ht2, ss, sd, se4, src4, dst4)


def kernel(x, edge_index, edge_attr, W_in, b_in, W_res, b_res, W_edge,
           W1, We1, a1, W2, We2, a2):
    ei32 = edge_index.astype(jnp.int32)
    src4 = ei32[0].reshape(_NS, _SEG, _SB, _RB)
    dst4 = ei32[1].reshape(_NS, _SEG, _SB, _RB)

    ht2_1, s1s, s1d = _node_prologue(
        x, W_in, b_in, W_res, b_res, W1,
        a1[:_D].reshape(_D, 1), a1[_D:2 * _D].reshape(_D, 1))
    e1, se1 = _edge_dense1(edge_attr, W_edge, We1, a1[2 * _D:].reshape(_D, 1))
    e2, se2 = _edge_dense2(e1, We2, a2[2 * _D:].reshape(_D, 1))

    wp1, dp1 = _sc_gat(ht2_1, s1s.reshape(_N), s1d.reshape(_N),
                       se1.reshape(_NS, _SEG, _SB, _RB), src4, dst4)
    ht2_2, s2s, s2d = _combine_mid(
        wp1, dp1.reshape(_N, 1), W2,
        a2[:_D].reshape(_D, 1), a2[_D:2 * _D].reshape(_D, 1))
    wp2, dp2 = _sc_gat(ht2_2, s2s.reshape(_N), s2d.reshape(_N),
                       se2.reshape(_NS, _SEG, _SB, _RB), src4, dst4)
    h = _combine_final(wp2, dp2.reshape(_N, 1))

    return h, edge_index, e2


# speedup vs baseline: 1.1715x; 1.1715x over previous
import functools

import jax
import jax.numpy as jnp
from jax import lax
from jax.experimental import pallas as pl
from jax.experimental.pallas import tpu as pltpu
from jax.experimental.pallas import tpu_sc as plsc

_N = 10000
_E = 320000
_D = 128
_HW = _D // 2
_NC = 2
_NS = 16
_RB = 80
_SEG = 5
_SB = 50
_RING = 5
_CH = 640
_CHL = _N - (_NS - 1) * _CH
_EB = 3200

_TC_PARAMS = pltpu.CompilerParams(vmem_limit_bytes=100 * 2**20)
_SC_PARAMS = pltpu.CompilerParams(needs_layout_passes=False,
                                  use_tc_tiling_on_sc=False)


def _row_dot(v, m):
    return lax.dot_general(v, m,
                           dimension_numbers=(((0,), (1,)), ((), ())),
                           preferred_element_type=jnp.float32)


def _node_prologue(x, W_in, b_in, W_res, b_res, W1, a1s, a1d):
    def body(x_ref, wi_ref, bi_ref, wr_ref, br_ref, w1_ref, as_ref, ad_ref,
             ht2_ref, ss_ref, sd_ref):
        h = jnp.dot(x_ref[...], wi_ref[...],
                    preferred_element_type=jnp.float32) + bi_ref[...]
        h = h + jnp.maximum(
            jnp.dot(h, wr_ref[...], preferred_element_type=jnp.float32)
            + br_ref[...], 0.0)
        ht = jnp.dot(h, w1_ref[...], preferred_element_type=jnp.float32)
        ht2_ref[pl.ds(0, _N), :] = ht[:, :_HW]
        ht2_ref[pl.ds(_N, _N), :] = ht[:, _HW:]
        ss_ref[...] = _row_dot(as_ref[...], ht)
        sd_ref[...] = _row_dot(ad_ref[...], ht)

    return pl.pallas_call(
        body,
        out_shape=[jax.ShapeDtypeStruct((2 * _N, _HW), jnp.float32),
                   jax.ShapeDtypeStruct((1, _N), jnp.float32),
                   jax.ShapeDtypeStruct((1, _N), jnp.float32)],
        compiler_params=_TC_PARAMS,
    )(x, W_in, b_in.reshape(1, _D), W_res, b_res.reshape(1, _D), W1, a1s, a1d)


def _edge_dense1(edge_attr, W_edge, We1, v1):
    def body(ea_ref, we_ref, w1_ref, v1_ref, s1_ref):
        ea0 = jnp.maximum(
            jnp.dot(ea_ref[...], we_ref[...], preferred_element_type=jnp.float32),
            0.0)
        v1p = jnp.dot(w1_ref[...], v1_ref[...], preferred_element_type=jnp.float32)
        s1_ref[...] = _row_dot(v1p, ea0).reshape(1, 1, _EB)

    full = lambda i: (0, 0)
    blk = lambda i: (i, 0)
    return pl.pallas_call(
        body,
        grid=(_E // _EB,),
        in_specs=[pl.BlockSpec((_EB, 16), blk),
                  pl.BlockSpec((16, _D), full),
                  pl.BlockSpec((_D, _D), full),
                  pl.BlockSpec((_D, 1), full)],
        out_specs=pl.BlockSpec((1, 1, _EB), lambda i: (i, 0, 0)),
        out_shape=jax.ShapeDtypeStruct((_E // _EB, 1, _EB), jnp.float32),
        compiler_params=_TC_PARAMS,
    )(edge_attr, W_edge, We1, v1)


def _edge_dense2(edge_attr, W_edge, We1, We2, v2):
    def body(ea_ref, we_ref, w1_ref, w2_ref, v2_ref, e2_ref, s2_ref):
        ea0 = jnp.maximum(
            jnp.dot(ea_ref[...], we_ref[...], preferred_element_type=jnp.float32),
            0.0)
        e1 = jnp.dot(ea0, w1_ref[...], preferred_element_type=jnp.float32)
        v2p = jnp.dot(w2_ref[...], v2_ref[...], preferred_element_type=jnp.float32)
        s2_ref[...] = _row_dot(v2p, e1).reshape(1, 1, _EB)
        e2_ref[...] = jnp.dot(e1, w2_ref[...], preferred_element_type=jnp.float32)

    full = lambda i: (0, 0)
    blk = lambda i: (i, 0)
    return pl.pallas_call(
        body,
        grid=(_E // _EB,),
        in_specs=[pl.BlockSpec((_EB, 16), blk),
                  pl.BlockSpec((16, _D), full),
                  pl.BlockSpec((_D, _D), full),
                  pl.BlockSpec((_D, _D), full),
                  pl.BlockSpec((_D, 1), full)],
        out_specs=[pl.BlockSpec((_EB, _D), blk),
                   pl.BlockSpec((1, 1, _EB), lambda i: (i, 0, 0))],
        out_shape=[jax.ShapeDtypeStruct((_E, _D), jnp.float32),
                   jax.ShapeDtypeStruct((_E // _EB, 1, _EB), jnp.float32)],
        compiler_params=_TC_PARAMS,
    )(edge_attr, W_edge, We1, We2, v2)


def _elu(x):
    return jnp.where(x > 0, x, jnp.exp(jnp.minimum(x, 0.0)) - 1.0)


def _combine_mid(wp, dp, W2, a2s, a2d):
    def body(wp_ref, dp_ref, w2_ref, as_ref, ad_ref, ht2_ref, ss_ref, sd_ref):
        w = jnp.concatenate([wp_ref[0], wp_ref[1]], axis=1)
        h1 = _elu(w / (dp_ref[...] + 1e-16))
        ht = jnp.dot(h1, w2_ref[...], preferred_element_type=jnp.float32)
        ht2_ref[pl.ds(0, _N), :] = ht[:, :_HW]
        ht2_ref[pl.ds(_N, _N), :] = ht[:, _HW:]
        ss_ref[...] = _row_dot(as_ref[...], ht)
        sd_ref[...] = _row_dot(ad_ref[...], ht)

    return pl.pallas_call(
        body,
        out_shape=[jax.ShapeDtypeStruct((2 * _N, _HW), jnp.float32),
                   jax.ShapeDtypeStruct((1, _N), jnp.float32),
                   jax.ShapeDtypeStruct((1, _N), jnp.float32)],
        compiler_params=_TC_PARAMS,
    )(wp, dp, W2, a2s, a2d)


def _combine_final(wp, dp):
    def body(wp_ref, dp_ref, h_ref):
        w = jnp.concatenate([wp_ref[0], wp_ref[1]], axis=1)
        h_ref[...] = _elu(w / (dp_ref[...] + 1e-16))

    return pl.pallas_call(
        body,
        out_shape=jax.ShapeDtypeStruct((_N, _D), jnp.float32),
        compiler_params=_TC_PARAMS,
    )(wp, dp)


def _sc_gat(ht2, ss, sd, se4, src4, dst4):
    mesh = plsc.VectorSubcoreMesh(core_axis_name="c", subcore_axis_name="s")

    @functools.partial(
        pl.kernel,
        out_type=[jax.ShapeDtypeStruct((_NC, _N, _HW), jnp.float32),
                  jax.ShapeDtypeStruct((_N,), jnp.float32)],
        mesh=mesh,
        compiler_params=_SC_PARAMS,
        scratch_types=(
            [pltpu.VMEM((_N,), jnp.float32),
             pltpu.VMEM((_N,), jnp.float32),
             pltpu.VMEM((_SB, _RB), jnp.int32),
             pltpu.VMEM((_SB, _RB), jnp.int32),
             pltpu.VMEM((_SB, _RB), jnp.float32),
             pltpu.VMEM((_SB, _RB), jnp.float32)]
            + [pltpu.VMEM((_RB, _HW), jnp.float32)] * _RING
            + [pltpu.VMEM_SHARED((_N, _HW), jnp.float32),
               pltpu.VMEM_SHARED((_N,), jnp.float32)]
            + [pltpu.SemaphoreType.DMA] * (2 * _RING)),
    )
    def k(ht_hbm, ss_hbm, sd_hbm, se_hbm, src_hbm, dst_hbm,
          wsum_hbm, den_hbm,
          ss_tab, sd_tab, src_seg, dst_seg, se_seg, w_seg,
          g0, g1, g2, g3, g4, wsum_sh, den_sh,
          gs0, gs1, gs2, gs3, gs4, ts0, ts1, ts2, ts3, ts4):
        c = lax.axis_index("c")
        s = lax.axis_index("s")
        gbufs = (g0, g1, g2, g3, g4)
        gsems = (gs0, gs1, gs2, gs3, gs4)
        ssems = (ts0, ts1, ts2, ts3, ts4)

        row0 = pl.multiple_of(s * _CH, 8)

        def _per_range(fn):
            @pl.when(s < _NS - 1)
            def _():
                fn(_CH)

            @pl.when(s == _NS - 1)
            def _():
                fn(_CHL)

        z16 = jnp.zeros((16,), jnp.float32)

        @pl.loop(0, _RB)
        def _(j):
            for q in range(_HW // 16):
                g0[j, pl.ds(q * 16, 16)] = z16

        def _zero(n):
            for t in range(n // _RB):
                pltpu.sync_copy(g0, wsum_sh.at[pl.ds(row0 + t * _RB, _RB)])

                @pl.when(c == 0)
                def _():
                    pltpu.sync_copy(g0.at[0, pl.ds(0, _RB)],
                                    den_sh.at[pl.ds(row0 + t * _RB, _RB)])

        _per_range(_zero)
        pltpu.sync_copy(ss_hbm, ss_tab)
        pltpu.sync_copy(sd_hbm, sd_tab)
        plsc.subcore_barrier()

        off = c * _N

        @pl.loop(0, _SEG)
        def _(seg):
            pltpu.sync_copy(src_hbm.at[s, seg], src_seg)
            pltpu.sync_copy(dst_hbm.at[s, seg], dst_seg)
            pltpu.sync_copy(se_hbm.at[s, seg], se_seg)

            @pl.loop(0, _SB)
            def _(b):
                for g in range(_RB // 16):
                    sl = pl.ds(g * 16, 16)
                    si = src_seg[b, sl]
                    di = dst_seg[b, sl]
                    vs = plsc.load_gather(ss_tab, [si])
                    vd = plsc.load_gather(sd_tab, [di])
                    l = vs + vd + se_seg[b, sl]
                    l = jnp.where(l >= 0.0, l, 0.2 * l)
                    w_seg[b, sl] = jnp.exp(l)
                    src_seg[b, sl] = si + off

                @pl.when(c == 0)
                def _():
                    pltpu.sync_copy(w_seg.at[b], den_sh.at[dst_seg.at[b]],
                                    add=True)

            for p in range(_RING - 1):
                pltpu.async_copy(ht_hbm.at[src_seg.at[p]], gbufs[p], gsems[p])

            @pl.loop(0, _SB, step=_RING)
            def _(r):
                for p in range(_RING):
                    rr = r + p
                    gbuf, gsem, ssem = gbufs[p], gsems[p], ssems[p]
                    pm = (p + _RING - 1) % _RING
                    pltpu.make_async_copy(ht_hbm.at[src_seg.at[rr]],
                                          gbuf, gsem).wait()

                    @plsc.parallel_loop(0, _RB // 16)
                    def _(j16):
                        w16 = w_seg[rr, pl.ds(j16 * 16, 16)]
                        for jj in range(16):
                            wj = w16[jj]
                            row = j16 * 16 + jj
                            for q in range(_HW // 16):
                                sl2 = pl.ds(q * 16, 16)
                                gbuf[row, sl2] = gbuf[row, sl2] * wj

                    pltpu.async_copy(gbuf, wsum_sh.at[dst_seg.at[rr]],
                                     ssem, add=True)

                    @pl.when(rr >= 1)
                    def _():
                        pltpu.make_async_copy(
                            gbufs[pm], wsum_sh.at[dst_seg.at[rr - 1]],
                            ssems[pm]).wait()

                    @pl.when(rr + _RING - 1 < _SB)
                    def _():
                        pltpu.async_copy(ht_hbm.at[src_seg.at[rr + _RING - 1]],
                                         gbufs[pm], gsems[pm])

            pltpu.make_async_copy(gbufs[(_SB - 1) % _RING],
                                  wsum_sh.at[dst_seg.at[_SB - 1]],
                                  ssems[(_SB - 1) % _RING]).wait()

        plsc.subcore_barrier()

        def _publish(n):
            for t in range(n // _RB):
                gb = gbufs[t % 2]
                r0 = row0 + t * _RB
                pltpu.sync_copy(wsum_sh.at[pl.ds(r0, _RB)], gb)
                pltpu.sync_copy(gb, wsum_hbm.at[c, pl.ds(r0, _RB)])

                @pl.when(c == 0)
                def _():
                    pltpu.sync_copy(den_sh.at[pl.ds(r0, _RB)],
                                    gb.at[0, pl.ds(0, _RB)])
                    pltpu.sync_copy(gb.at[0, pl.ds(0, _RB)],
                                    den_hbm.at[pl.ds(r0, _RB)])

        _per_range(_publish)

    return k(ht2, ss, sd, se4, src4, dst4)


def kernel(x, edge_index, edge_attr, W_in, b_in, W_res, b_res, W_edge,
           W1, We1, a1, W2, We2, a2):
    ei32 = edge_index.astype(jnp.int32)
    src4 = ei32[0].reshape(_NS, _SEG, _SB, _RB)
    dst4 = ei32[1].reshape(_NS, _SEG, _SB, _RB)

    ht2_1, s1s, s1d = _node_prologue(
        x, W_in, b_in, W_res, b_res, W1,
        a1[:_D].reshape(_D, 1), a1[_D:2 * _D].reshape(_D, 1))
    se1 = _edge_dense1(edge_attr, W_edge, We1, a1[2 * _D:].reshape(_D, 1))
    e2, se2 = _edge_dense2(edge_attr, W_edge, We1, We2,
                           a2[2 * _D:].reshape(_D, 1))

    wp1, dp1 = _sc_gat(ht2_1, s1s.reshape(_N), s1d.reshape(_N),
                       se1.reshape(_NS, _SEG, _SB, _RB), src4, dst4)
    ht2_2, s2s, s2d = _combine_mid(
        wp1, dp1.reshape(_N, 1), W2,
        a2[:_D].reshape(_D, 1), a2[_D:2 * _D].reshape(_D, 1))
    wp2, dp2 = _sc_gat(ht2_2, s2s.reshape(_N), s2d.reshape(_N),
                       se2.reshape(_NS, _SEG, _SB, _RB), src4, dst4)
    h = _combine_final(wp2, dp2.reshape(_N, 1))

    return h, edge_index, e2


# speedup vs baseline: 1.4148x; 1.2077x over previous
import functools

import jax
import jax.numpy as jnp
from jax import lax
from jax.experimental import pallas as pl
from jax.experimental.pallas import tpu as pltpu
from jax.experimental.pallas import tpu_sc as plsc

_N = 10000
_E = 320000
_D = 128
_HW = _D // 2
_NC = 2
_NS = 16
_RB = 80
_SEG = 5
_SB = 50
_RING = 5
_CH = 640
_CHL = _N - (_NS - 1) * _CH
_EB = 3200

_TC_PARAMS = pltpu.CompilerParams(vmem_limit_bytes=100 * 2**20)
_SC_PARAMS = pltpu.CompilerParams(needs_layout_passes=False,
                                  use_tc_tiling_on_sc=False)


def _row_dot(v, m):
    return lax.dot_general(v, m,
                           dimension_numbers=(((0,), (1,)), ((), ())),
                           preferred_element_type=jnp.float32)


def _tdot(at, b):
    return lax.dot_general(at, b, dimension_numbers=(((0,), (0,)), ((), ())),
                           preferred_element_type=jnp.float32)


def _node_prologue(xt, W_in, b_in, W_res, b_res, W1, a1s, a1d):
    def body(x_ref, wi_ref, bi_ref, wr_ref, br_ref, w1_ref, as_ref, ad_ref,
             ht2_ref, ss_ref, sd_ref):
        h = _tdot(x_ref[...], wi_ref[...]) + bi_ref[...]
        h = h + jnp.maximum(
            jnp.dot(h, wr_ref[...], preferred_element_type=jnp.float32)
            + br_ref[...], 0.0)
        ht = jnp.dot(h, w1_ref[...], preferred_element_type=jnp.float32)
        ht2_ref[pl.ds(0, _N), :] = ht[:, :_HW]
        ht2_ref[pl.ds(_N, _N), :] = ht[:, _HW:]
        ss_ref[...] = _row_dot(as_ref[...], ht)
        sd_ref[...] = _row_dot(ad_ref[...], ht)

    return pl.pallas_call(
        body,
        out_shape=[jax.ShapeDtypeStruct((2 * _N, _HW), jnp.float32),
                   jax.ShapeDtypeStruct((1, _N), jnp.float32),
                   jax.ShapeDtypeStruct((1, _N), jnp.float32)],
        compiler_params=_TC_PARAMS,
    )(xt, W_in, b_in.reshape(1, _D), W_res, b_res.reshape(1, _D), W1, a1s, a1d)


def _edge_dense1(eat, W_edge, We1, v1):
    def body(ea_ref, we_ref, w1_ref, v1_ref, s1_ref):
        ea0 = jnp.maximum(_tdot(ea_ref[...], we_ref[...]), 0.0)
        v1p = jnp.dot(w1_ref[...], v1_ref[...], preferred_element_type=jnp.float32)
        s1_ref[...] = _row_dot(v1p, ea0).reshape(1, 1, _EB)

    full = lambda i: (0, 0)
    return pl.pallas_call(
        body,
        grid=(_E // _EB,),
        in_specs=[pl.BlockSpec((16, _EB), lambda i: (0, i)),
                  pl.BlockSpec((16, _D), full),
                  pl.BlockSpec((_D, _D), full),
                  pl.BlockSpec((_D, 1), full)],
        out_specs=pl.BlockSpec((1, 1, _EB), lambda i: (i, 0, 0)),
        out_shape=jax.ShapeDtypeStruct((_E // _EB, 1, _EB), jnp.float32),
        compiler_params=_TC_PARAMS,
    )(eat, W_edge, We1, v1)


def _edge_dense2(eat, W_edge, We1, We2, v2):
    def body(ea_ref, we_ref, w1_ref, w2_ref, v2_ref, e2_ref, s2_ref):
        ea0 = jnp.maximum(_tdot(ea_ref[...], we_ref[...]), 0.0)
        e1 = jnp.dot(ea0, w1_ref[...], preferred_element_type=jnp.float32)
        v2p = jnp.dot(w2_ref[...], v2_ref[...], preferred_element_type=jnp.float32)
        s2_ref[...] = _row_dot(v2p, e1).reshape(1, 1, _EB)
        e2_ref[...] = jnp.dot(e1, w2_ref[...], preferred_element_type=jnp.float32)

    full = lambda i: (0, 0)
    blk = lambda i: (i, 0)
    return pl.pallas_call(
        body,
        grid=(_E // _EB,),
        in_specs=[pl.BlockSpec((16, _EB), lambda i: (0, i)),
                  pl.BlockSpec((16, _D), full),
                  pl.BlockSpec((_D, _D), full),
                  pl.BlockSpec((_D, _D), full),
                  pl.BlockSpec((_D, 1), full)],
        out_specs=[pl.BlockSpec((_EB, _D), blk),
                   pl.BlockSpec((1, 1, _EB), lambda i: (i, 0, 0))],
        out_shape=[jax.ShapeDtypeStruct((_E, _D), jnp.float32),
                   jax.ShapeDtypeStruct((_E // _EB, 1, _EB), jnp.float32)],
        compiler_params=_TC_PARAMS,
    )(eat, W_edge, We1, We2, v2)


def _elu(x):
    return jnp.where(x > 0, x, jnp.exp(jnp.minimum(x, 0.0)) - 1.0)


def _combine_mid(wp, dp, W2, a2s, a2d):
    def body(wp_ref, dp_ref, w2_ref, as_ref, ad_ref, ht2_ref, ss_ref, sd_ref):
        w = jnp.concatenate([wp_ref[0], wp_ref[1]], axis=1)
        h1 = _elu(w / (dp_ref[...] + 1e-16))
        ht = jnp.dot(h1, w2_ref[...], preferred_element_type=jnp.float32)
        ht2_ref[pl.ds(0, _N), :] = ht[:, :_HW]
        ht2_ref[pl.ds(_N, _N), :] = ht[:, _HW:]
        ss_ref[...] = _row_dot(as_ref[...], ht)
        sd_ref[...] = _row_dot(ad_ref[...], ht)

    return pl.pallas_call(
        body,
        out_shape=[jax.ShapeDtypeStruct((2 * _N, _HW), jnp.float32),
                   jax.ShapeDtypeStruct((1, _N), jnp.float32),
                   jax.ShapeDtypeStruct((1, _N), jnp.float32)],
        compiler_params=_TC_PARAMS,
    )(wp, dp, W2, a2s, a2d)


def _combine_final(wp, dp):
    def body(wp_ref, dp_ref, h_ref):
        w = jnp.concatenate([wp_ref[0], wp_ref[1]], axis=1)
        h_ref[...] = _elu(w / (dp_ref[...] + 1e-16))

    return pl.pallas_call(
        body,
        out_shape=jax.ShapeDtypeStruct((_N, _D), jnp.float32),
        compiler_params=_TC_PARAMS,
    )(wp, dp)


def _sc_gat(ht2, ss, sd, se4, src4, dst4):
    mesh = plsc.VectorSubcoreMesh(core_axis_name="c", subcore_axis_name="s")

    @functools.partial(
        pl.kernel,
        out_type=[jax.ShapeDtypeStruct((_NC, _N, _HW), jnp.float32),
                  jax.ShapeDtypeStruct((_N,), jnp.float32)],
        mesh=mesh,
        compiler_params=_SC_PARAMS,
        scratch_types=(
            [pltpu.VMEM((_N,), jnp.float32),
             pltpu.VMEM((_N,), jnp.float32),
             pltpu.VMEM((_SB, _RB), jnp.int32),
             pltpu.VMEM((_SB, _RB), jnp.int32),
             pltpu.VMEM((_SB, _RB), jnp.float32),
             pltpu.VMEM((_SB, _RB), jnp.float32)]
            + [pltpu.VMEM((_RB, _HW), jnp.float32)] * _RING
            + [pltpu.VMEM_SHARED((_N, _HW), jnp.float32),
               pltpu.VMEM_SHARED((_N,), jnp.float32)]
            + [pltpu.SemaphoreType.DMA] * (2 * _RING)),
    )
    def k(ht_hbm, ss_hbm, sd_hbm, se_hbm, src_hbm, dst_hbm,
          wsum_hbm, den_hbm,
          ss_tab, sd_tab, src_seg, dst_seg, se_seg, w_seg,
          g0, g1, g2, g3, g4, wsum_sh, den_sh,
          gs0, gs1, gs2, gs3, gs4, ts0, ts1, ts2, ts3, ts4):
        c = lax.axis_index("c")
        s = lax.axis_index("s")
        gbufs = (g0, g1, g2, g3, g4)
        gsems = (gs0, gs1, gs2, gs3, gs4)
        ssems = (ts0, ts1, ts2, ts3, ts4)

        row0 = pl.multiple_of(s * _CH, 8)

        def _per_range(fn):
            @pl.when(s < _NS - 1)
            def _():
                fn(_CH)

            @pl.when(s == _NS - 1)
            def _():
                fn(_CHL)

        z16 = jnp.zeros((16,), jnp.float32)

        @pl.loop(0, _RB)
        def _(j):
            for q in range(_HW // 16):
                g0[j, pl.ds(q * 16, 16)] = z16

        def _zero(n):
            for t in range(n // _RB):
                pltpu.sync_copy(g0, wsum_sh.at[pl.ds(row0 + t * _RB, _RB)])

                @pl.when(c == 0)
                def _():
                    pltpu.sync_copy(g0.at[0, pl.ds(0, _RB)],
                                    den_sh.at[pl.ds(row0 + t * _RB, _RB)])

        _per_range(_zero)
        pltpu.sync_copy(ss_hbm, ss_tab)
        pltpu.sync_copy(sd_hbm, sd_tab)
        plsc.subcore_barrier()

        off = c * _N

        @pl.loop(0, _SEG)
        def _(seg):
            pltpu.sync_copy(src_hbm.at[s, seg], src_seg)
            pltpu.sync_copy(dst_hbm.at[s, seg], dst_seg)
            pltpu.sync_copy(se_hbm.at[s, seg], se_seg)

            @pl.loop(0, _SB)
            def _(b):
                for g in range(_RB // 16):
                    sl = pl.ds(g * 16, 16)
                    si = src_seg[b, sl]
                    di = dst_seg[b, sl]
                    vs = plsc.load_gather(ss_tab, [si])
                    vd = plsc.load_gather(sd_tab, [di])
                    l = vs + vd + se_seg[b, sl]
                    l = jnp.where(l >= 0.0, l, 0.2 * l)
                    w_seg[b, sl] = jnp.exp(l)
                    src_seg[b, sl] = si + off

                @pl.when(c == 0)
                def _():
                    pltpu.sync_copy(w_seg.at[b], den_sh.at[dst_seg.at[b]],
                                    add=True)

            for p in range(_RING - 1):
                pltpu.async_copy(ht_hbm.at[src_seg.at[p]], gbufs[p], gsems[p])

            @pl.loop(0, _SB, step=_RING)
            def _(r):
                for p in range(_RING):
                    rr = r + p
                    gbuf, gsem, ssem = gbufs[p], gsems[p], ssems[p]
                    pm = (p + _RING - 1) % _RING
                    pltpu.make_async_copy(ht_hbm.at[src_seg.at[rr]],
                                          gbuf, gsem).wait()

                    @plsc.parallel_loop(0, _RB // 16)
                    def _(j16):
                        w16 = w_seg[rr, pl.ds(j16 * 16, 16)]
                        for jj in range(16):
                            wj = w16[jj]
                            row = j16 * 16 + jj
                            for q in range(_HW // 16):
                                sl2 = pl.ds(q * 16, 16)
                                gbuf[row, sl2] = gbuf[row, sl2] * wj

                    pltpu.async_copy(gbuf, wsum_sh.at[dst_seg.at[rr]],
                                     ssem, add=True)

                    @pl.when(rr >= 1)
                    def _():
                        pltpu.make_async_copy(
                            gbufs[pm], wsum_sh.at[dst_seg.at[rr - 1]],
                            ssems[pm]).wait()

                    @pl.when(rr + _RING - 1 < _SB)
                    def _():
                        pltpu.async_copy(ht_hbm.at[src_seg.at[rr + _RING - 1]],
                                         gbufs[pm], gsems[pm])

            pltpu.make_async_copy(gbufs[(_SB - 1) % _RING],
                                  wsum_sh.at[dst_seg.at[_SB - 1]],
                                  ssems[(_SB - 1) % _RING]).wait()

        plsc.subcore_barrier()

        def _publish(n):
            for t in range(n // _RB):
                gb = gbufs[t % 2]
                r0 = row0 + t * _RB
                pltpu.sync_copy(wsum_sh.at[pl.ds(r0, _RB)], gb)
                pltpu.sync_copy(gb, wsum_hbm.at[c, pl.ds(r0, _RB)])

                @pl.when(c == 0)
                def _():
                    pltpu.sync_copy(den_sh.at[pl.ds(r0, _RB)],
                                    gb.at[0, pl.ds(0, _RB)])
                    pltpu.sync_copy(gb.at[0, pl.ds(0, _RB)],
                                    den_hbm.at[pl.ds(r0, _RB)])

        _per_range(_publish)

    return k(ht2, ss, sd, se4, src4, dst4)


def kernel(x, edge_index, edge_attr, W_in, b_in, W_res, b_res, W_edge,
           W1, We1, a1, W2, We2, a2):
    ei32 = edge_index.astype(jnp.int32)
    src4 = ei32[0].reshape(_NS, _SEG, _SB, _RB)
    dst4 = ei32[1].reshape(_NS, _SEG, _SB, _RB)

    xt = x.T
    eat = edge_attr.T
    ht2_1, s1s, s1d = _node_prologue(
        xt, W_in, b_in, W_res, b_res, W1,
        a1[:_D].reshape(_D, 1), a1[_D:2 * _D].reshape(_D, 1))
    se1 = _edge_dense1(eat, W_edge, We1, a1[2 * _D:].reshape(_D, 1))
    e2, se2 = _edge_dense2(eat, W_edge, We1, We2,
                           a2[2 * _D:].reshape(_D, 1))

    wp1, dp1 = _sc_gat(ht2_1, s1s.reshape(_N), s1d.reshape(_N),
                       se1.reshape(_NS, _SEG, _SB, _RB), src4, dst4)
    ht2_2, s2s, s2d = _combine_mid(
        wp1, dp1.reshape(_N, 1), W2,
        a2[:_D].reshape(_D, 1), a2[_D:2 * _D].reshape(_D, 1))
    wp2, dp2 = _sc_gat(ht2_2, s2s.reshape(_N), s2d.reshape(_N),
                       se2.reshape(_NS, _SEG, _SB, _RB), src4, dst4)
    h = _combine_final(wp2, dp2.reshape(_N, 1))

    return h, edge_index, e2


# speedup vs baseline: 1.4194x; 1.0032x over previous
import functools

import jax
import jax.numpy as jnp
from jax import lax
from jax.experimental import pallas as pl
from jax.experimental.pallas import tpu as pltpu
from jax.experimental.pallas import tpu_sc as plsc

_N = 10000
_E = 320000
_D = 128
_HW = _D // 2
_NC = 2
_NS = 16
_RB = 80
_SEG = 5
_SB = 50
_RING = 5
_CH = 640
_CHL = _N - (_NS - 1) * _CH
_EB = 3200

_TC_PARAMS = pltpu.CompilerParams(vmem_limit_bytes=100 * 2**20)
_SC_PARAMS = pltpu.CompilerParams(needs_layout_passes=False,
                                  use_tc_tiling_on_sc=False)


def _row_dot(v, m):
    return lax.dot_general(v, m,
                           dimension_numbers=(((0,), (1,)), ((), ())),
                           preferred_element_type=jnp.float32)


def _tdot(at, b):
    return lax.dot_general(at, b, dimension_numbers=(((0,), (0,)), ((), ())),
                           preferred_element_type=jnp.float32)


def _node_prologue(xt, W_in, b_in, W_res, b_res, W1, a1s, a1d):
    def body(x_ref, wi_ref, bi_ref, wr_ref, br_ref, w1_ref, as_ref, ad_ref,
             ht2_ref, ss_ref, sd_ref):
        h = _tdot(x_ref[...], wi_ref[...]) + bi_ref[...]
        h = h + jnp.maximum(
            jnp.dot(h, wr_ref[...], preferred_element_type=jnp.float32)
            + br_ref[...], 0.0)
        ht = jnp.dot(h, w1_ref[...], preferred_element_type=jnp.float32)
        ht2_ref[pl.ds(0, _N), :] = ht[:, :_HW]
        ht2_ref[pl.ds(_N, _N), :] = ht[:, _HW:]
        ss_ref[...] = _row_dot(as_ref[...], ht)
        sd_ref[...] = _row_dot(ad_ref[...], ht)

    return pl.pallas_call(
        body,
        out_shape=[jax.ShapeDtypeStruct((2 * _N, _HW), jnp.float32),
                   jax.ShapeDtypeStruct((1, _N), jnp.float32),
                   jax.ShapeDtypeStruct((1, _N), jnp.float32)],
        compiler_params=_TC_PARAMS,
    )(xt, W_in, b_in.reshape(1, _D), W_res, b_res.reshape(1, _D), W1, a1s, a1d)


def _edge_dense1(eat, W_edge, We1, v1):
    def body(ea_ref, we_ref, w1_ref, v1_ref, s1_ref):
        ea0 = jnp.maximum(
            lax.dot_general(ea_ref[...].astype(jnp.bfloat16),
                            we_ref[...].astype(jnp.bfloat16),
                            dimension_numbers=(((0,), (0,)), ((), ())),
                            preferred_element_type=jnp.float32), 0.0)
        v1p = jnp.dot(w1_ref[...], v1_ref[...], preferred_element_type=jnp.float32)
        s1_ref[...] = _row_dot(v1p, ea0).reshape(1, 1, _EB)

    full = lambda i: (0, 0)
    return pl.pallas_call(
        body,
        grid=(_E // _EB,),
        in_specs=[pl.BlockSpec((16, _EB), lambda i: (0, i)),
                  pl.BlockSpec((16, _D), full),
                  pl.BlockSpec((_D, _D), full),
                  pl.BlockSpec((_D, 1), full)],
        out_specs=pl.BlockSpec((1, 1, _EB), lambda i: (i, 0, 0)),
        out_shape=jax.ShapeDtypeStruct((_E // _EB, 1, _EB), jnp.float32),
        compiler_params=_TC_PARAMS,
    )(eat, W_edge, We1, v1)


def _edge_dense2(eat, W_edge, We1, We2, v2):
    def body(ea_ref, we_ref, w1_ref, w2_ref, v2_ref, e2_ref, s2_ref):
        ea0 = jnp.maximum(_tdot(ea_ref[...], we_ref[...]), 0.0)
        e1 = jnp.dot(ea0, w1_ref[...], preferred_element_type=jnp.float32)
        v2p = jnp.dot(w2_ref[...], v2_ref[...], preferred_element_type=jnp.float32)
        s2_ref[...] = _row_dot(v2p, e1).reshape(1, 1, _EB)
        e2_ref[...] = jnp.dot(e1, w2_ref[...], preferred_element_type=jnp.float32)

    full = lambda i: (0, 0)
    blk = lambda i: (i, 0)
    return pl.pallas_call(
        body,
        grid=(_E // _EB,),
        in_specs=[pl.BlockSpec((16, _EB), lambda i: (0, i)),
                  pl.BlockSpec((16, _D), full),
                  pl.BlockSpec((_D, _D), full),
                  pl.BlockSpec((_D, _D), full),
                  pl.BlockSpec((_D, 1), full)],
        out_specs=[pl.BlockSpec((_EB, _D), blk),
                   pl.BlockSpec((1, 1, _EB), lambda i: (i, 0, 0))],
        out_shape=[jax.ShapeDtypeStruct((_E, _D), jnp.float32),
                   jax.ShapeDtypeStruct((_E // _EB, 1, _EB), jnp.float32)],
        compiler_params=_TC_PARAMS,
    )(eat, W_edge, We1, We2, v2)


def _elu(x):
    return jnp.where(x > 0, x, jnp.exp(jnp.minimum(x, 0.0)) - 1.0)


def _combine_mid(wp, dp, W2, a2s, a2d):
    def body(wp_ref, dp_ref, w2_ref, as_ref, ad_ref, ht2_ref, ss_ref, sd_ref):
        w = jnp.concatenate([wp_ref[0], wp_ref[1]], axis=1)
        h1 = _elu(w / (dp_ref[...] + 1e-16))
        ht = jnp.dot(h1, w2_ref[...], preferred_element_type=jnp.float32)
        ht2_ref[pl.ds(0, _N), :] = ht[:, :_HW]
        ht2_ref[pl.ds(_N, _N), :] = ht[:, _HW:]
        ss_ref[...] = _row_dot(as_ref[...], ht)
        sd_ref[...] = _row_dot(ad_ref[...], ht)

    return pl.pallas_call(
        body,
        out_shape=[jax.ShapeDtypeStruct((2 * _N, _HW), jnp.float32),
                   jax.ShapeDtypeStruct((1, _N), jnp.float32),
                   jax.ShapeDtypeStruct((1, _N), jnp.float32)],
        compiler_params=_TC_PARAMS,
    )(wp, dp, W2, a2s, a2d)


def _combine_final(wp, dp):
    def body(wp_ref, dp_ref, h_ref):
        w = jnp.concatenate([wp_ref[0], wp_ref[1]], axis=1)
        h_ref[...] = _elu(w / (dp_ref[...] + 1e-16))

    return pl.pallas_call(
        body,
        out_shape=jax.ShapeDtypeStruct((_N, _D), jnp.float32),
        compiler_params=_TC_PARAMS,
    )(wp, dp)


def _sc_gat(ht2, ss, sd, se4, src4, dst4):
    mesh = plsc.VectorSubcoreMesh(core_axis_name="c", subcore_axis_name="s")

    @functools.partial(
        pl.kernel,
        out_type=[jax.ShapeDtypeStruct((_NC, _N, _HW), jnp.float32),
                  jax.ShapeDtypeStruct((_N,), jnp.float32)],
        mesh=mesh,
        compiler_params=_SC_PARAMS,
        scratch_types=(
            [pltpu.VMEM((_N,), jnp.float32),
             pltpu.VMEM((_N,), jnp.float32),
             pltpu.VMEM((_SB, _RB), jnp.int32),
             pltpu.VMEM((_SB, _RB), jnp.int32),
             pltpu.VMEM((_SB, _RB), jnp.float32),
             pltpu.VMEM((_SB, _RB), jnp.float32)]
            + [pltpu.VMEM((_RB, _HW), jnp.float32)] * _RING
            + [pltpu.VMEM_SHARED((_N, _HW), jnp.float32),
               pltpu.VMEM_SHARED((_N,), jnp.float32)]
            + [pltpu.SemaphoreType.DMA] * (2 * _RING)),
    )
    def k(ht_hbm, ss_hbm, sd_hbm, se_hbm, src_hbm, dst_hbm,
          wsum_hbm, den_hbm,
          ss_tab, sd_tab, src_seg, dst_seg, se_seg, w_seg,
          g0, g1, g2, g3, g4, wsum_sh, den_sh,
          gs0, gs1, gs2, gs3, gs4, ts0, ts1, ts2, ts3, ts4):
        c = lax.axis_index("c")
        s = lax.axis_index("s")
        gbufs = (g0, g1, g2, g3, g4)
        gsems = (gs0, gs1, gs2, gs3, gs4)
        ssems = (ts0, ts1, ts2, ts3, ts4)

        row0 = pl.multiple_of(s * _CH, 8)

        def _per_range(fn):
            @pl.when(s < _NS - 1)
            def _():
                fn(_CH)

            @pl.when(s == _NS - 1)
            def _():
                fn(_CHL)

        z16 = jnp.zeros((16,), jnp.float32)

        @pl.loop(0, _RB)
        def _(j):
            for q in range(_HW // 16):
                g0[j, pl.ds(q * 16, 16)] = z16

        def _zero(n):
            for t in range(n // _RB):
                pltpu.sync_copy(g0, wsum_sh.at[pl.ds(row0 + t * _RB, _RB)])

                @pl.when(c == 0)
                def _():
                    pltpu.sync_copy(g0.at[0, pl.ds(0, _RB)],
                                    den_sh.at[pl.ds(row0 + t * _RB, _RB)])

        _per_range(_zero)
        pltpu.sync_copy(ss_hbm, ss_tab)
        pltpu.sync_copy(sd_hbm, sd_tab)
        plsc.subcore_barrier()

        off = c * _N

        @pl.loop(0, _SEG)
        def _(seg):
            pltpu.sync_copy(src_hbm.at[s, seg], src_seg)
            pltpu.sync_copy(dst_hbm.at[s, seg], dst_seg)
            pltpu.sync_copy(se_hbm.at[s, seg], se_seg)

            @pl.loop(0, _SB)
            def _(b):
                for g in range(_RB // 16):
                    sl = pl.ds(g * 16, 16)
                    si = src_seg[b, sl]
                    di = dst_seg[b, sl]
                    vs = plsc.load_gather(ss_tab, [si])
                    vd = plsc.load_gather(sd_tab, [di])
                    l = vs + vd + se_seg[b, sl]
                    l = jnp.where(l >= 0.0, l, 0.2 * l)
                    w_seg[b, sl] = jnp.exp(l)
                    src_seg[b, sl] = si + off

                @pl.when(c == 0)
                def _():
                    pltpu.sync_copy(w_seg.at[b], den_sh.at[dst_seg.at[b]],
                                    add=True)

            for p in range(_RING - 1):
                pltpu.async_copy(ht_hbm.at[src_seg.at[p]], gbufs[p], gsems[p])

            @pl.loop(0, _SB, step=_RING)
            def _(r):
                for p in range(_RING):
                    rr = r + p
                    gbuf, gsem, ssem = gbufs[p], gsems[p], ssems[p]
                    pm = (p + _RING - 1) % _RING
                    pltpu.make_async_copy(ht_hbm.at[src_seg.at[rr]],
                                          gbuf, gsem).wait()

                    @plsc.parallel_loop(0, _RB // 16)
                    def _(j16):
                        w16 = w_seg[rr, pl.ds(j16 * 16, 16)]
                        for jj in range(16):
                            wj = w16[jj]
                            row = j16 * 16 + jj
                            for q in range(_HW // 16):
                                sl2 = pl.ds(q * 16, 16)
                                gbuf[row, sl2] = gbuf[row, sl2] * wj

                    pltpu.async_copy(gbuf, wsum_sh.at[dst_seg.at[rr]],
                                     ssem, add=True)

                    @pl.when(rr >= 1)
                    def _():
                        pltpu.make_async_copy(
                            gbufs[pm], wsum_sh.at[dst_seg.at[rr - 1]],
                            ssems[pm]).wait()

                    @pl.when(rr + _RING - 1 < _SB)
                    def _():
                        pltpu.async_copy(ht_hbm.at[src_seg.at[rr + _RING - 1]],
                                         gbufs[pm], gsems[pm])

            pltpu.make_async_copy(gbufs[(_SB - 1) % _RING],
                                  wsum_sh.at[dst_seg.at[_SB - 1]],
                                  ssems[(_SB - 1) % _RING]).wait()

        plsc.subcore_barrier()

        def _publish(n):
            for t in range(n // _RB):
                gb = gbufs[t % 2]
                r0 = row0 + t * _RB
                pltpu.sync_copy(wsum_sh.at[pl.ds(r0, _RB)], gb)
                pltpu.sync_copy(gb, wsum_hbm.at[c, pl.ds(r0, _RB)])

                @pl.when(c == 0)
                def _():
                    pltpu.sync_copy(den_sh.at[pl.ds(r0, _RB)],
                                    gb.at[0, pl.ds(0, _RB)])
                    pltpu.sync_copy(gb.at[0, pl.ds(0, _RB)],
                                    den_hbm.at[pl.ds(r0, _RB)])

        _per_range(_publish)

    return k(ht2, ss, sd, se4, src4, dst4)


def kernel(x, edge_index, edge_attr, W_in, b_in, W_res, b_res, W_edge,
           W1, We1, a1, W2, We2, a2):
    ei32 = edge_index.astype(jnp.int32)
    src4 = ei32[0].reshape(_NS, _SEG, _SB, _RB)
    dst4 = ei32[1].reshape(_NS, _SEG, _SB, _RB)

    xt = x.T
    eat = edge_attr.T
    ht2_1, s1s, s1d = _node_prologue(
        xt, W_in, b_in, W_res, b_res, W1,
        a1[:_D].reshape(_D, 1), a1[_D:2 * _D].reshape(_D, 1))
    se1 = _edge_dense1(eat, W_edge, We1, a1[2 * _D:].reshape(_D, 1))
    e2, se2 = _edge_dense2(eat, W_edge, We1, We2,
                           a2[2 * _D:].reshape(_D, 1))

    wp1, dp1 = _sc_gat(ht2_1, s1s.reshape(_N), s1d.reshape(_N),
                       se1.reshape(_NS, _SEG, _SB, _RB), src4, dst4)
    ht2_2, s2s, s2d = _combine_mid(
        wp1, dp1.reshape(_N, 1), W2,
        a2[:_D].reshape(_D, 1), a2[_D:2 * _D].reshape(_D, 1))
    wp2, dp2 = _sc_gat(ht2_2, s2s.reshape(_N), s2d.reshape(_N),
                       se2.reshape(_NS, _SEG, _SB, _RB), src4, dst4)
    h = _combine_final(wp2, dp2.reshape(_N, 1))

    return h, edge_index, e2


# speedup vs baseline: 1.5058x; 1.0609x over previous
import functools

import jax
import jax.numpy as jnp
from jax import lax
from jax.experimental import pallas as pl
from jax.experimental.pallas import tpu as pltpu
from jax.experimental.pallas import tpu_sc as plsc

_N = 10000
_E = 320000
_D = 128
_HW = _D // 2
_NC = 2
_NS = 16
_RB = 80
_SEG = 5
_SB = 50
_RING = 5
_CH = 640
_CHL = _N - (_NS - 1) * _CH
_EB = 3200

_TC_PARAMS = pltpu.CompilerParams(vmem_limit_bytes=100 * 2**20)
_SC_PARAMS = pltpu.CompilerParams(needs_layout_passes=False,
                                  use_tc_tiling_on_sc=False)


def _row_dot(v, m):
    return lax.dot_general(v, m,
                           dimension_numbers=(((0,), (1,)), ((), ())),
                           preferred_element_type=jnp.float32)


def _tdot(at, b):
    return lax.dot_general(at, b, dimension_numbers=(((0,), (0,)), ((), ())),
                           preferred_element_type=jnp.float32)


def _node_prologue(xt, W_in, b_in, W_res, b_res, W1, a1s, a1d):
    def body(x_ref, wi_ref, bi_ref, wr_ref, br_ref, w1_ref, as_ref, ad_ref,
             ht2_ref, ss_ref, sd_ref):
        h = _tdot(x_ref[...], wi_ref[...]) + bi_ref[...]
        h = h + jnp.maximum(
            jnp.dot(h, wr_ref[...], preferred_element_type=jnp.float32)
            + br_ref[...], 0.0)
        ht = jnp.dot(h, w1_ref[...], preferred_element_type=jnp.float32)
        ht2_ref[pl.ds(0, _N), :] = ht[:, :_HW]
        ht2_ref[pl.ds(_N, _N), :] = ht[:, _HW:]
        ss_ref[...] = _row_dot(as_ref[...], ht)
        sd_ref[...] = _row_dot(ad_ref[...], ht)

    return pl.pallas_call(
        body,
        out_shape=[jax.ShapeDtypeStruct((2 * _N, _HW), jnp.float32),
                   jax.ShapeDtypeStruct((1, _N), jnp.float32),
                   jax.ShapeDtypeStruct((1, _N), jnp.float32)],
        compiler_params=_TC_PARAMS,
    )(xt, W_in, b_in.reshape(1, _D), W_res, b_res.reshape(1, _D), W1, a1s, a1d)


def _edge_dense1(eat, W_edge, We1, v1):
    def body(ea_ref, we_ref, w1_ref, v1_ref, s1_ref):
        ea0 = jnp.maximum(_tdot(ea_ref[...], we_ref[...]), 0.0)
        v1p = jnp.dot(w1_ref[...], v1_ref[...], preferred_element_type=jnp.float32)
        s1_ref[...] = _row_dot(v1p, ea0).reshape(1, 1, _EB)

    full = lambda i: (0, 0)
    return pl.pallas_call(
        body,
        grid=(_E // _EB,),
        in_specs=[pl.BlockSpec((16, _EB), lambda i: (0, i)),
                  pl.BlockSpec((16, _D), full),
                  pl.BlockSpec((_D, _D), full),
                  pl.BlockSpec((_D, 1), full)],
        out_specs=pl.BlockSpec((1, 1, _EB), lambda i: (i, 0, 0)),
        out_shape=jax.ShapeDtypeStruct((_E // _EB, 1, _EB), jnp.float32),
        compiler_params=_TC_PARAMS,
    )(eat, W_edge, We1, v1)


def _edge_dense2(eat, W_edge, We1, We2, v2):
    def body(ea_ref, we_ref, w1_ref, w2_ref, v2_ref, e2_ref, s2_ref):
        ea0 = jnp.maximum(_tdot(ea_ref[...], we_ref[...]), 0.0)
        e1 = jnp.dot(ea0, w1_ref[...], preferred_element_type=jnp.float32)
        v2p = jnp.dot(w2_ref[...], v2_ref[...], preferred_element_type=jnp.float32)
        s2_ref[...] = _row_dot(v2p, e1).reshape(1, 1, _EB)
        e2_ref[...] = jnp.dot(e1, w2_ref[...], preferred_element_type=jnp.float32)

    full = lambda i: (0, 0)
    blk = lambda i: (i, 0)
    return pl.pallas_call(
        body,
        grid=(_E // _EB,),
        in_specs=[pl.BlockSpec((16, _EB), lambda i: (0, i)),
                  pl.BlockSpec((16, _D), full),
                  pl.BlockSpec((_D, _D), full),
                  pl.BlockSpec((_D, _D), full),
                  pl.BlockSpec((_D, 1), full)],
        out_specs=[pl.BlockSpec((_EB, _D), blk),
                   pl.BlockSpec((1, 1, _EB), lambda i: (i, 0, 0))],
        out_shape=[jax.ShapeDtypeStruct((_E, _D), jnp.float32),
                   jax.ShapeDtypeStruct((_E // _EB, 1, _EB), jnp.float32)],
        compiler_params=_TC_PARAMS,
    )(eat, W_edge, We1, We2, v2)


def _elu(x):
    return jnp.where(x > 0, x, jnp.exp(jnp.minimum(x, 0.0)) - 1.0)


def _combine_mid(wp, dp, W2, a2s, a2d):
    def body(wp_ref, dp_ref, w2_ref, as_ref, ad_ref, ht2_ref, ss_ref, sd_ref):
        w = jnp.concatenate([wp_ref[0], wp_ref[1]], axis=1)
        h1 = _elu(w / (dp_ref[...] + 1e-16))
        ht = jnp.dot(h1, w2_ref[...], preferred_element_type=jnp.float32)
        ht2_ref[pl.ds(0, _N), :] = ht[:, :_HW]
        ht2_ref[pl.ds(_N, _N), :] = ht[:, _HW:]
        ss_ref[...] = _row_dot(as_ref[...], ht)
        sd_ref[...] = _row_dot(ad_ref[...], ht)

    return pl.pallas_call(
        body,
        out_shape=[jax.ShapeDtypeStruct((2 * _N, _HW), jnp.float32),
                   jax.ShapeDtypeStruct((1, _N), jnp.float32),
                   jax.ShapeDtypeStruct((1, _N), jnp.float32)],
        compiler_params=_TC_PARAMS,
    )(wp, dp, W2, a2s, a2d)


def _combine_final(wp, dp):
    def body(wp_ref, dp_ref, h_ref):
        w = jnp.concatenate([wp_ref[0], wp_ref[1]], axis=1)
        h_ref[...] = _elu(w / (dp_ref[...] + 1e-16))

    return pl.pallas_call(
        body,
        out_shape=jax.ShapeDtypeStruct((_N, _D), jnp.float32),
        compiler_params=_TC_PARAMS,
    )(wp, dp)


def _sc_gat(ht2, ss, sd, se4, src4, dst4):
    mesh = plsc.VectorSubcoreMesh(core_axis_name="c", subcore_axis_name="s")

    @functools.partial(
        pl.kernel,
        out_type=[jax.ShapeDtypeStruct((_NC, _N, _HW), jnp.float32),
                  jax.ShapeDtypeStruct((_N,), jnp.float32)],
        mesh=mesh,
        compiler_params=_SC_PARAMS,
        scratch_types=(
            [pltpu.VMEM((_N,), jnp.float32),
             pltpu.VMEM((_N,), jnp.float32),
             pltpu.VMEM((_SB, _RB), jnp.int32),
             pltpu.VMEM((_SB, _RB), jnp.int32),
             pltpu.VMEM((_SB, _RB), jnp.float32),
             pltpu.VMEM((_SB, _RB), jnp.float32)]
            + [pltpu.VMEM((_RB, _HW), jnp.float32)] * _RING
            + [pltpu.VMEM_SHARED((_N, _HW), jnp.float32),
               pltpu.VMEM_SHARED((_N,), jnp.float32)]
            + [pltpu.SemaphoreType.DMA] * (2 * _RING + 1)),
    )
    def k(ht_hbm, ss_hbm, sd_hbm, se_hbm, src_hbm, dst_hbm,
          wsum_hbm, den_hbm,
          ss_tab, sd_tab, src_seg, dst_seg, se_seg, w_seg,
          g0, g1, g2, g3, g4, wsum_sh, den_sh,
          gs0, gs1, gs2, gs3, gs4, ts0, ts1, ts2, ts3, ts4, dsem):
        c = lax.axis_index("c")
        s = lax.axis_index("s")
        gbufs = (g0, g1, g2, g3, g4)
        gsems = (gs0, gs1, gs2, gs3, gs4)
        ssems = (ts0, ts1, ts2, ts3, ts4)

        row0 = pl.multiple_of(s * _CH, 8)

        def _per_range(fn):
            @pl.when(s < _NS - 1)
            def _():
                fn(_CH)

            @pl.when(s == _NS - 1)
            def _():
                fn(_CHL)

        z16 = jnp.zeros((16,), jnp.float32)

        @pl.loop(0, _RB)
        def _(j):
            for q in range(_HW // 16):
                g0[j, pl.ds(q * 16, 16)] = z16

        def _zero(n):
            for t in range(n // _RB):
                pltpu.sync_copy(g0, wsum_sh.at[pl.ds(row0 + t * _RB, _RB)])

                @pl.when(c == 0)
                def _():
                    pltpu.sync_copy(g0.at[0, pl.ds(0, _RB)],
                                    den_sh.at[pl.ds(row0 + t * _RB, _RB)])

        _per_range(_zero)
        pltpu.sync_copy(ss_hbm, ss_tab)
        pltpu.sync_copy(sd_hbm, sd_tab)
        plsc.subcore_barrier()

        off = c * _N

        @pl.loop(0, _SEG)
        def _(seg):
            pltpu.sync_copy(src_hbm.at[s, seg], src_seg)
            pltpu.sync_copy(dst_hbm.at[s, seg], dst_seg)
            pltpu.sync_copy(se_hbm.at[s, seg], se_seg)

            @pl.loop(0, _SB)
            def _(b):
                for g in range(_RB // 16):
                    sl = pl.ds(g * 16, 16)
                    si = src_seg[b, sl]
                    di = dst_seg[b, sl]
                    vs = plsc.load_gather(ss_tab, [si])
                    vd = plsc.load_gather(sd_tab, [di])
                    l = vs + vd + se_seg[b, sl]
                    l = jnp.where(l >= 0.0, l, 0.2 * l)
                    w_seg[b, sl] = jnp.exp(l)
                    src_seg[b, sl] = si + off

                @pl.when(c == 0)
                def _():
                    pltpu.async_copy(w_seg.at[b], den_sh.at[dst_seg.at[b]],
                                     dsem, add=True)

            for p in range(_RING - 1):
                pltpu.async_copy(ht_hbm.at[src_seg.at[p]], gbufs[p], gsems[p])

            @pl.loop(0, _SB, step=_RING)
            def _(r):
                for p in range(_RING):
                    rr = r + p
                    gbuf, gsem, ssem = gbufs[p], gsems[p], ssems[p]
                    pm = (p + _RING - 1) % _RING
                    pltpu.make_async_copy(ht_hbm.at[src_seg.at[rr]],
                                          gbuf, gsem).wait()

                    @plsc.parallel_loop(0, _RB // 16)
                    def _(j16):
                        w16 = w_seg[rr, pl.ds(j16 * 16, 16)]
                        for jj in range(16):
                            wj = w16[jj]
                            row = j16 * 16 + jj
                            for q in range(_HW // 16):
                                sl2 = pl.ds(q * 16, 16)
                                gbuf[row, sl2] = gbuf[row, sl2] * wj

                    pltpu.async_copy(gbuf, wsum_sh.at[dst_seg.at[rr]],
                                     ssem, add=True)

                    @pl.when(rr >= 1)
                    def _():
                        pltpu.make_async_copy(
                            gbufs[pm], wsum_sh.at[dst_seg.at[rr - 1]],
                            ssems[pm]).wait()

                    @pl.when(rr + _RING - 1 < _SB)
                    def _():
                        pltpu.async_copy(ht_hbm.at[src_seg.at[rr + _RING - 1]],
                                         gbufs[pm], gsems[pm])

            pltpu.make_async_copy(gbufs[(_SB - 1) % _RING],
                                  wsum_sh.at[dst_seg.at[_SB - 1]],
                                  ssems[(_SB - 1) % _RING]).wait()

            @pl.when(c == 0)
            def _():
                pltpu.make_async_copy(se_hbm.at[s, seg], w_seg, dsem).wait()

        plsc.subcore_barrier()

        def _publish(n):
            for t in range(n // _RB):
                gb = gbufs[t % 2]
                r0 = row0 + t * _RB
                pltpu.sync_copy(wsum_sh.at[pl.ds(r0, _RB)], gb)
                pltpu.sync_copy(gb, wsum_hbm.at[c, pl.ds(r0, _RB)])

                @pl.when(c == 0)
                def _():
                    pltpu.sync_copy(den_sh.at[pl.ds(r0, _RB)],
                                    gb.at[0, pl.ds(0, _RB)])
                    pltpu.sync_copy(gb.at[0, pl.ds(0, _RB)],
                                    den_hbm.at[pl.ds(r0, _RB)])

        _per_range(_publish)

    return k(ht2, ss, sd, se4, src4, dst4)


def kernel(x, edge_index, edge_attr, W_in, b_in, W_res, b_res, W_edge,
           W1, We1, a1, W2, We2, a2):
    ei32 = edge_index.astype(jnp.int32)
    src4 = ei32[0].reshape(_NS, _SEG, _SB, _RB)
    dst4 = ei32[1].reshape(_NS, _SEG, _SB, _RB)

    xt = x.T
    eat = edge_attr.T
    ht2_1, s1s, s1d = _node_prologue(
        xt, W_in, b_in, W_res, b_res, W1,
        a1[:_D].reshape(_D, 1), a1[_D:2 * _D].reshape(_D, 1))
    se1 = _edge_dense1(eat, W_edge, We1, a1[2 * _D:].reshape(_D, 1))
    e2, se2 = _edge_dense2(eat, W_edge, We1, We2,
                           a2[2 * _D:].reshape(_D, 1))

    wp1, dp1 = _sc_gat(ht2_1, s1s.reshape(_N), s1d.reshape(_N),
                       se1.reshape(_NS, _SEG, _SB, _RB), src4, dst4)
    ht2_2, s2s, s2d = _combine_mid(
        wp1, dp1.reshape(_N, 1), W2,
        a2[:_D].reshape(_D, 1), a2[_D:2 * _D].reshape(_D, 1))
    wp2, dp2 = _sc_gat(ht2_2, s2s.reshape(_N), s2d.reshape(_N),
                       se2.reshape(_NS, _SEG, _SB, _RB), src4, dst4)
    h = _combine_final(wp2, dp2.reshape(_N, 1))

    return h, edge_index, e2


# speedup vs baseline: 1.5321x; 1.0174x over previous
import functools

import jax
import jax.numpy as jnp
from jax import lax
from jax.experimental import pallas as pl
from jax.experimental.pallas import tpu as pltpu
from jax.experimental.pallas import tpu_sc as plsc

_N = 10000
_E = 320000
_D = 128
_HW = _D // 2
_NC = 2
_NS = 16
_RB = 80
_SEG = 2
_SB = 125
_RING = 5
_CH = 640
_CHL = _N - (_NS - 1) * _CH
_EB = 3200

_TC_PARAMS = pltpu.CompilerParams(vmem_limit_bytes=100 * 2**20)
_SC_PARAMS = pltpu.CompilerParams(needs_layout_passes=False,
                                  use_tc_tiling_on_sc=False)


def _row_dot(v, m):
    return lax.dot_general(v, m,
                           dimension_numbers=(((0,), (1,)), ((), ())),
                           preferred_element_type=jnp.float32)


def _tdot(at, b):
    return lax.dot_general(at, b, dimension_numbers=(((0,), (0,)), ((), ())),
                           preferred_element_type=jnp.float32)


def _node_prologue(xt, W_in, b_in, W_res, b_res, W1, a1s, a1d):
    def body(x_ref, wi_ref, bi_ref, wr_ref, br_ref, w1_ref, as_ref, ad_ref,
             ht2_ref, ss_ref, sd_ref):
        h = _tdot(x_ref[...], wi_ref[...]) + bi_ref[...]
        h = h + jnp.maximum(
            jnp.dot(h, wr_ref[...], preferred_element_type=jnp.float32)
            + br_ref[...], 0.0)
        ht = jnp.dot(h, w1_ref[...], preferred_element_type=jnp.float32)
        ht2_ref[pl.ds(0, _N), :] = ht[:, :_HW]
        ht2_ref[pl.ds(_N, _N), :] = ht[:, _HW:]
        ss_ref[...] = _row_dot(as_ref[...], ht)
        sd_ref[...] = _row_dot(ad_ref[...], ht)

    return pl.pallas_call(
        body,
        out_shape=[jax.ShapeDtypeStruct((2 * _N, _HW), jnp.float32),
                   jax.ShapeDtypeStruct((1, _N), jnp.float32),
                   jax.ShapeDtypeStruct((1, _N), jnp.float32)],
        compiler_params=_TC_PARAMS,
    )(xt, W_in, b_in.reshape(1, _D), W_res, b_res.reshape(1, _D), W1, a1s, a1d)


def _edge_dense1(eat, W_edge, We1, v1):
    def body(ea_ref, we_ref, w1_ref, v1_ref, s1_ref):
        ea0 = jnp.maximum(_tdot(ea_ref[...], we_ref[...]), 0.0)
        v1p = jnp.dot(w1_ref[...], v1_ref[...], preferred_element_type=jnp.float32)
        s1_ref[...] = _row_dot(v1p, ea0).reshape(1, 1, _EB)

    full = lambda i: (0, 0)
    return pl.pallas_call(
        body,
        grid=(_E // _EB,),
        in_specs=[pl.BlockSpec((16, _EB), lambda i: (0, i)),
                  pl.BlockSpec((16, _D), full),
                  pl.BlockSpec((_D, _D), full),
                  pl.BlockSpec((_D, 1), full)],
        out_specs=pl.BlockSpec((1, 1, _EB), lambda i: (i, 0, 0)),
        out_shape=jax.ShapeDtypeStruct((_E // _EB, 1, _EB), jnp.float32),
        compiler_params=_TC_PARAMS,
    )(eat, W_edge, We1, v1)


def _edge_dense2(eat, W_edge, We1, We2, v2):
    def body(ea_ref, we_ref, w1_ref, w2_ref, v2_ref, e2_ref, s2_ref):
        ea0 = jnp.maximum(_tdot(ea_ref[...], we_ref[...]), 0.0)
        e1 = jnp.dot(ea0, w1_ref[...], preferred_element_type=jnp.float32)
        v2p = jnp.dot(w2_ref[...], v2_ref[...], preferred_element_type=jnp.float32)
        s2_ref[...] = _row_dot(v2p, e1).reshape(1, 1, _EB)
        e2_ref[...] = jnp.dot(e1, w2_ref[...], preferred_element_type=jnp.float32)

    full = lambda i: (0, 0)
    blk = lambda i: (i, 0)
    return pl.pallas_call(
        body,
        grid=(_E // _EB,),
        in_specs=[pl.BlockSpec((16, _EB), lambda i: (0, i)),
                  pl.BlockSpec((16, _D), full),
                  pl.BlockSpec((_D, _D), full),
                  pl.BlockSpec((_D, _D), full),
                  pl.BlockSpec((_D, 1), full)],
        out_specs=[pl.BlockSpec((_EB, _D), blk),
                   pl.BlockSpec((1, 1, _EB), lambda i: (i, 0, 0))],
        out_shape=[jax.ShapeDtypeStruct((_E, _D), jnp.float32),
                   jax.ShapeDtypeStruct((_E // _EB, 1, _EB), jnp.float32)],
        compiler_params=_TC_PARAMS,
    )(eat, W_edge, We1, We2, v2)


def _elu(x):
    return jnp.where(x > 0, x, jnp.exp(jnp.minimum(x, 0.0)) - 1.0)


def _combine_mid(wp, dp, W2, a2s, a2d):
    def body(wp_ref, dp_ref, w2_ref, as_ref, ad_ref, ht2_ref, ss_ref, sd_ref):
        w = jnp.concatenate([wp_ref[0], wp_ref[1]], axis=1)
        h1 = _elu(w / (dp_ref[...] + 1e-16))
        ht = jnp.dot(h1, w2_ref[...], preferred_element_type=jnp.float32)
        ht2_ref[pl.ds(0, _N), :] = ht[:, :_HW]
        ht2_ref[pl.ds(_N, _N), :] = ht[:, _HW:]
        ss_ref[...] = _row_dot(as_ref[...], ht)
        sd_ref[...] = _row_dot(ad_ref[...], ht)

    return pl.pallas_call(
        body,
        out_shape=[jax.ShapeDtypeStruct((2 * _N, _HW), jnp.float32),
                   jax.ShapeDtypeStruct((1, _N), jnp.float32),
                   jax.ShapeDtypeStruct((1, _N), jnp.float32)],
        compiler_params=_TC_PARAMS,
    )(wp, dp, W2, a2s, a2d)


def _combine_final(wp, dp):
    def body(wp_ref, dp_ref, h_ref):
        w = jnp.concatenate([wp_ref[0], wp_ref[1]], axis=1)
        h_ref[...] = _elu(w / (dp_ref[...] + 1e-16))

    return pl.pallas_call(
        body,
        out_shape=jax.ShapeDtypeStruct((_N, _D), jnp.float32),
        compiler_params=_TC_PARAMS,
    )(wp, dp)


def _sc_gat(ht2, ss, sd, se4, src4, dst4):
    mesh = plsc.VectorSubcoreMesh(core_axis_name="c", subcore_axis_name="s")

    @functools.partial(
        pl.kernel,
        out_type=[jax.ShapeDtypeStruct((_NC, _N, _HW), jnp.float32),
                  jax.ShapeDtypeStruct((_N,), jnp.float32)],
        mesh=mesh,
        compiler_params=_SC_PARAMS,
        scratch_types=(
            [pltpu.VMEM((_N,), jnp.float32),
             pltpu.VMEM((_N,), jnp.float32),
             pltpu.VMEM((_SB, _RB), jnp.int32),
             pltpu.VMEM((_SB, _RB), jnp.int32),
             pltpu.VMEM((_SB, _RB), jnp.float32),
             pltpu.VMEM((_SB, _RB), jnp.float32)]
            + [pltpu.VMEM((_RB, _HW), jnp.float32)] * _RING
            + [pltpu.VMEM_SHARED((_N, _HW), jnp.float32),
               pltpu.VMEM_SHARED((_N,), jnp.float32)]
            + [pltpu.SemaphoreType.DMA] * (2 * _RING + 1)),
    )
    def k(ht_hbm, ss_hbm, sd_hbm, se_hbm, src_hbm, dst_hbm,
          wsum_hbm, den_hbm,
          ss_tab, sd_tab, src_seg, dst_seg, se_seg, w_seg,
          g0, g1, g2, g3, g4, wsum_sh, den_sh,
          gs0, gs1, gs2, gs3, gs4, ts0, ts1, ts2, ts3, ts4, dsem):
        c = lax.axis_index("c")
        s = lax.axis_index("s")
        gbufs = (g0, g1, g2, g3, g4)
        gsems = (gs0, gs1, gs2, gs3, gs4)
        ssems = (ts0, ts1, ts2, ts3, ts4)

        row0 = pl.multiple_of(s * _CH, 8)

        def _per_range(fn):
            @pl.when(s < _NS - 1)
            def _():
                fn(_CH)

            @pl.when(s == _NS - 1)
            def _():
                fn(_CHL)

        z16 = jnp.zeros((16,), jnp.float32)

        @pl.loop(0, _RB)
        def _(j):
            for q in range(_HW // 16):
                g0[j, pl.ds(q * 16, 16)] = z16

        def _zero(n):
            for t in range(n // _RB):
                pltpu.sync_copy(g0, wsum_sh.at[pl.ds(row0 + t * _RB, _RB)])

                @pl.when(c == 0)
                def _():
                    pltpu.sync_copy(g0.at[0, pl.ds(0, _RB)],
                                    den_sh.at[pl.ds(row0 + t * _RB, _RB)])

        _per_range(_zero)
        pltpu.sync_copy(ss_hbm, ss_tab)
        pltpu.sync_copy(sd_hbm, sd_tab)
        plsc.subcore_barrier()

        off = c * _N

        @pl.loop(0, _SEG)
        def _(seg):
            pltpu.sync_copy(src_hbm.at[s, seg], src_seg)
            pltpu.sync_copy(dst_hbm.at[s, seg], dst_seg)
            pltpu.sync_copy(se_hbm.at[s, seg], se_seg)

            @pl.loop(0, _SB)
            def _(b):
                for g in range(_RB // 16):
                    sl = pl.ds(g * 16, 16)
                    si = src_seg[b, sl]
                    di = dst_seg[b, sl]
                    vs = plsc.load_gather(ss_tab, [si])
                    vd = plsc.load_gather(sd_tab, [di])
                    l = vs + vd + se_seg[b, sl]
                    l = jnp.where(l >= 0.0, l, 0.2 * l)
                    w_seg[b, sl] = jnp.exp(l)
                    src_seg[b, sl] = si + off

                @pl.when(c == 0)
                def _():
                    pltpu.async_copy(w_seg.at[b], den_sh.at[dst_seg.at[b]],
                                     dsem, add=True)

            for p in range(_RING - 1):
                pltpu.async_copy(ht_hbm.at[src_seg.at[p]], gbufs[p], gsems[p])

            @pl.loop(0, _SB, step=_RING)
            def _(r):
                for p in range(_RING):
                    rr = r + p
                    gbuf, gsem, ssem = gbufs[p], gsems[p], ssems[p]
                    pm = (p + _RING - 1) % _RING
                    pltpu.make_async_copy(ht_hbm.at[src_seg.at[rr]],
                                          gbuf, gsem).wait()

                    @plsc.parallel_loop(0, _RB // 16)
                    def _(j16):
                        w16 = w_seg[rr, pl.ds(j16 * 16, 16)]
                        for jj in range(16):
                            wj = w16[jj]
                            row = j16 * 16 + jj
                            for q in range(_HW // 16):
                                sl2 = pl.ds(q * 16, 16)
                                gbuf[row, sl2] = gbuf[row, sl2] * wj

                    pltpu.async_copy(gbuf, wsum_sh.at[dst_seg.at[rr]],
                                     ssem, add=True)

                    @pl.when(rr >= 1)
                    def _():
                        pltpu.make_async_copy(
                            gbufs[pm], wsum_sh.at[dst_seg.at[rr - 1]],
                            ssems[pm]).wait()

                    @pl.when(rr + _RING - 1 < _SB)
                    def _():
                        pltpu.async_copy(ht_hbm.at[src_seg.at[rr + _RING - 1]],
                                         gbufs[pm], gsems[pm])

            pltpu.make_async_copy(gbufs[(_SB - 1) % _RING],
                                  wsum_sh.at[dst_seg.at[_SB - 1]],
                                  ssems[(_SB - 1) % _RING]).wait()

            @pl.when(c == 0)
            def _():
                pltpu.make_async_copy(se_hbm.at[s, seg], w_seg, dsem).wait()

        plsc.subcore_barrier()

        def _publish(n):
            for t in range(n // _RB):
                gb = gbufs[t % 2]
                r0 = row0 + t * _RB
                pltpu.sync_copy(wsum_sh.at[pl.ds(r0, _RB)], gb)
                pltpu.sync_copy(gb, wsum_hbm.at[c, pl.ds(r0, _RB)])

                @pl.when(c == 0)
                def _():
                    pltpu.sync_copy(den_sh.at[pl.ds(r0, _RB)],
                                    gb.at[0, pl.ds(0, _RB)])
                    pltpu.sync_copy(gb.at[0, pl.ds(0, _RB)],
                                    den_hbm.at[pl.ds(r0, _RB)])

        _per_range(_publish)

    return k(ht2, ss, sd, se4, src4, dst4)


def kernel(x, edge_index, edge_attr, W_in, b_in, W_res, b_res, W_edge,
           W1, We1, a1, W2, We2, a2):
    ei32 = edge_index.astype(jnp.int32)
    src4 = ei32[0].reshape(_NS, _SEG, _SB, _RB)
    dst4 = ei32[1].reshape(_NS, _SEG, _SB, _RB)

    xt = x.T
    eat = edge_attr.T
    ht2_1, s1s, s1d = _node_prologue(
        xt, W_in, b_in, W_res, b_res, W1,
        a1[:_D].reshape(_D, 1), a1[_D:2 * _D].reshape(_D, 1))
    se1 = _edge_dense1(eat, W_edge, We1, a1[2 * _D:].reshape(_D, 1))
    e2, se2 = _edge_dense2(eat, W_edge, We1, We2,
                           a2[2 * _D:].reshape(_D, 1))

    wp1, dp1 = _sc_gat(ht2_1, s1s.reshape(_N), s1d.reshape(_N),
                       se1.reshape(_NS, _SEG, _SB, _RB), src4, dst4)
    ht2_2, s2s, s2d = _combine_mid(
        wp1, dp1.reshape(_N, 1), W2,
        a2[:_D].reshape(_D, 1), a2[_D:2 * _D].reshape(_D, 1))
    wp2, dp2 = _sc_gat(ht2_2, s2s.reshape(_N), s2d.reshape(_N),
                       se2.reshape(_NS, _SEG, _SB, _RB), src4, dst4)
    h = _combine_final(wp2, dp2.reshape(_N, 1))

    return h, edge_index, e2
